# baseline (speedup 1.0000x reference)
import functools

import jax
import jax.numpy as jnp
from jax.experimental import pallas as pl
from jax.experimental.pallas import tpu as pltpu

_LANES = 128
_PAD0 = 3
_PAD1 = 2
_PAD2 = 1


def _fcn_kernel(x_ref, w0_ref, t0_ref, w1_ref, t1_ref, w2_ref, t2_ref,
                fcw_ref, fcb_ref, o_ref):
    B, Lp, _ = x_ref.shape
    L = Lp - 8

    xb = x_ref[...]
    cols0 = jnp.concatenate([xb[:, k:k + L, :] for k in range(7)], axis=-1)
    a0 = jnp.dot(cols0.reshape(B * L, 14), w0_ref[...],
                 preferred_element_type=jnp.float32)
    h0 = jnp.maximum(a0 + t0_ref[...], 0.0).astype(jnp.bfloat16)

    h0p = jnp.pad(h0.reshape(B, L, _LANES),
                  ((0, 0), (_PAD1, _PAD1), (0, 0)))
    z1 = jnp.dot(h0p.reshape(B * (L + 4), _LANES), w1_ref[...],
                 preferred_element_type=jnp.float32)
    z1 = z1.reshape(B, L + 4, 5 * _LANES)
    a1 = z1[:, 0:L, 0:_LANES]
    for k in range(1, 5):
        a1 = a1 + z1[:, k:k + L, k * _LANES:(k + 1) * _LANES]
    h1 = jnp.maximum(a1 + t1_ref[...], 0.0).astype(jnp.bfloat16)

    h1p = jnp.pad(h1, ((0, 0), (_PAD2, _PAD2), (0, 0)))
    z2 = jnp.dot(h1p.reshape(B * (L + 2), _LANES), w2_ref[...],
                 preferred_element_type=jnp.float32)
    z2 = z2.reshape(B, L + 2, 3 * _LANES)
    a2 = z2[:, 0:L, 0:_LANES]
    for k in range(1, 3):
        a2 = a2 + z2[:, k:k + L, k * _LANES:(k + 1) * _LANES]
    h2 = jnp.maximum(a2 + t2_ref[...], 0.0)

    pooled = jnp.mean(h2, axis=1)
    out = jnp.dot(pooled, fcw_ref[...], preferred_element_type=jnp.float32)
    o_ref[0] = out + fcb_ref[...]


def _fold(w, scale):
    return (w * scale[None, None, :]).astype(jnp.float32)


def kernel(conv0_w, conv0_scale, conv0_shift, conv1_w, conv1_scale,
           conv1_shift, conv2_w, conv2_scale, conv2_shift, fc_w, fc_b, x):
    N, cin, L = x.shape
    b_tile = 16
    num_tiles = pl.cdiv(N, b_tile)
    n_pad = num_tiles * b_tile

    xt = jnp.transpose(x, (0, 2, 1))
    xp = jnp.pad(xt, ((0, n_pad - N), (_PAD0, 8 - 2 * _PAD0 + _PAD0), (0, 0)))
    xp = xp.astype(jnp.bfloat16)

    w0 = _fold(conv0_w, conv0_scale)
    w0 = jnp.pad(w0, ((0, 0), (0, 0), (0, _LANES - w0.shape[2])))
    w0 = w0.reshape(14, _LANES).astype(jnp.bfloat16)
    t0 = jnp.pad(conv0_shift, (0, _LANES - conv0_shift.shape[0])).reshape(1, _LANES)

    w1 = _fold(conv1_w, conv1_scale)
    w1 = jnp.pad(w1, ((0, 0), (0, _LANES - w1.shape[1]), (0, 0)))
    w1 = jnp.transpose(w1, (1, 0, 2)).reshape(_LANES, 5 * _LANES)
    w1 = w1.astype(jnp.bfloat16)
    t1 = conv1_shift.reshape(1, _LANES)

    w2 = _fold(conv2_w, conv2_scale)
    w2 = jnp.pad(w2, ((0, 0), (0, 0), (0, _LANES - w2.shape[2])))
    w2 = jnp.transpose(w2, (1, 0, 2)).reshape(_LANES, 3 * _LANES)
    w2 = w2.astype(jnp.bfloat16)
    t2 = jnp.pad(conv2_shift, (0, _LANES - conv2_shift.shape[0])).reshape(1, _LANES)

    fcw = jnp.pad(fc_w, ((0, _LANES - fc_w.shape[0]), (0, 0)))
    fcb = fc_b.reshape(1, 2)

    consts = [w0, t0, w1, t1, w2, t2, fcw, fcb]
    Lp = L + 8
    out = pl.pallas_call(
        _fcn_kernel,
        out_shape=jax.ShapeDtypeStruct((num_tiles, b_tile, 2), jnp.float32),
        grid=(num_tiles,),
        in_specs=[pl.BlockSpec((b_tile, Lp, 2), lambda n: (n, 0, 0))]
        + [pl.BlockSpec(a.shape, lambda n, nd=a.ndim: (0,) * nd) for a in consts],
        out_specs=pl.BlockSpec((1, b_tile, 2), lambda n: (n, 0, 0)),
        compiler_params=pltpu.CompilerParams(
            dimension_semantics=("parallel",)),
    )(xp, *consts)
    return out.reshape(n_pad, 2)[:N]


# speedup vs baseline: 3.0915x; 3.0915x over previous
import jax
import jax.numpy as jnp
from jax.experimental import pallas as pl
from jax.experimental.pallas import tpu as pltpu

_LANES = 128
_SLOT = 136
_GAP = 4


def _lshift(h, s):
    if s == 0:
        return h
    if s > 0:
        return jnp.pad(h[:, s:], ((0, 0), (0, s)))
    return jnp.pad(h[:, :s], ((0, 0), (-s, 0)))


def _fcn_kernel(x_ref, w0_ref, t0_ref, w1_ref, t1_ref, w2_ref, t2_ref,
                pool_ref, fcw_ref, fcb_ref, o_ref):
    NL = x_ref.shape[1]
    pos = jax.lax.broadcasted_iota(jnp.int32, (1, NL), 1) % _SLOT
    live = (pos >= _GAP) & (pos < _SLOT - _GAP)

    xb = jnp.pad(x_ref[...], ((0, 16 - x_ref.shape[0]), (0, 0)))
    cols0 = jnp.concatenate([_lshift(xb, k - 3) for k in range(7)], axis=0)
    a0 = jnp.dot(w0_ref[...], cols0, preferred_element_type=jnp.float32)
    h0 = jnp.where(live, jnp.maximum(a0 + t0_ref[...], 0.0),
                   0.0).astype(jnp.bfloat16)

    cols1 = jnp.concatenate([_lshift(h0, k - 2) for k in range(5)], axis=0)
    a1 = jnp.dot(w1_ref[...], cols1, preferred_element_type=jnp.float32)
    h1 = jnp.where(live, jnp.maximum(a1 + t1_ref[...], 0.0),
                   0.0).astype(jnp.bfloat16)

    cols2 = jnp.concatenate([_lshift(h1, k - 1) for k in range(3)], axis=0)
    a2 = jnp.dot(w2_ref[...], cols2, preferred_element_type=jnp.float32)
    h2 = jnp.maximum(a2 + t2_ref[...], 0.0).astype(jnp.bfloat16)

    pooled = jnp.dot(h2, pool_ref[...], preferred_element_type=jnp.float32)
    out = jax.lax.dot_general(pooled, fcw_ref[...], (((0,), (0,)), ((), ())),
                              preferred_element_type=jnp.float32)
    o_ref[0] = out + fcb_ref[...]


def kernel(conv0_w, conv0_scale, conv0_shift, conv1_w, conv1_scale,
           conv1_shift, conv2_w, conv2_scale, conv2_shift, fc_w, fc_b, x):
    N, cin, L = x.shape
    b_tile = 16
    num_tiles = pl.cdiv(N, b_tile)
    n_pad = num_tiles * b_tile
    NL = b_tile * _SLOT

    xt = jnp.transpose(x, (1, 0, 2))
    xt = jnp.pad(xt, ((0, 0), (0, n_pad - N), (_GAP, _GAP)))
    xg = xt.reshape(cin, n_pad * _SLOT).astype(jnp.bfloat16)

    w0 = conv0_w * conv0_scale[None, None, :]
    w0 = jnp.pad(w0, ((0, 0), (0, 16 - cin), (0, _LANES - w0.shape[2])))
    w0 = jnp.transpose(w0, (2, 0, 1)).reshape(_LANES, 7 * 16)
    w0 = w0.astype(jnp.bfloat16)
    t0 = jnp.pad(conv0_shift, (0, _LANES - conv0_shift.shape[0])).reshape(_LANES, 1)

    w1 = conv1_w * conv1_scale[None, None, :]
    w1 = jnp.pad(w1, ((0, 0), (0, _LANES - w1.shape[1]), (0, 0)))
    w1 = jnp.transpose(w1, (2, 0, 1)).reshape(_LANES, 5 * _LANES)
    w1 = w1.astype(jnp.bfloat16)
    t1 = conv1_shift.reshape(_LANES, 1)

    w2 = conv2_w * conv2_scale[None, None, :]
    w2 = jnp.transpose(w2, (2, 0, 1)).reshape(64, 3 * _LANES)
    w2 = w2.astype(jnp.bfloat16)
    t2 = conv2_shift.reshape(64, 1)

    ar = jnp.arange(NL)
    posv = ar % _SLOT
    sig = (posv >= _GAP) & (posv < _SLOT - _GAP)
    blk = ar // _SLOT
    pm = (sig[:, None] & (blk[:, None] == jnp.arange(b_tile)[None, :]))
    pm = (pm.astype(jnp.float32) / L).astype(jnp.bfloat16)

    fcw = jnp.pad(fc_w, ((0, 0), (0, 0))).astype(jnp.float32)
    fcb = fc_b.reshape(1, 2)

    consts = [w0, t0, w1, t1, w2, t2, pm, fcw, fcb]
    out = pl.pallas_call(
        _fcn_kernel,
        out_shape=jax.ShapeDtypeStruct((num_tiles, b_tile, 2), jnp.float32),
        grid=(num_tiles,),
        in_specs=[pl.BlockSpec((cin, NL), lambda n: (0, n))]
        + [pl.BlockSpec(a.shape, lambda n, nd=a.ndim: (0,) * nd) for a in consts],
        out_specs=pl.BlockSpec((1, b_tile, 2), lambda n: (n, 0, 0)),
        compiler_params=pltpu.CompilerParams(
            dimension_semantics=("parallel",)),
    )(xg, *consts)
    return out.reshape(n_pad, 2)[:N]


# speedup vs baseline: 3.6219x; 1.1715x over previous
import jax
import jax.numpy as jnp
from jax.experimental import pallas as pl
from jax.experimental.pallas import tpu as pltpu

_LANES = 128
_SLOT = 136
_GAP = 4


def _lshift(h, s):
    if s == 0:
        return h
    if s > 0:
        return jnp.pad(h[:, s:], ((0, 0), (0, s)))
    return jnp.pad(h[:, :s], ((0, 0), (-s, 0)))


def _fcn_kernel(x_ref, w0_ref, t0_ref, w1_ref, t1_ref, w2_ref, t2_ref,
                pool_ref, fcw_ref, fcb_ref, o_ref):
    NL = x_ref.shape[1]
    pos = jax.lax.broadcasted_iota(jnp.int32, (1, NL), 1) % _SLOT
    live = (pos >= _GAP) & (pos < _SLOT - _GAP)

    xb = jnp.pad(x_ref[...], ((0, 16 - x_ref.shape[0]), (0, 0)))
    cols0 = jnp.concatenate([_lshift(xb, k - 3) for k in range(7)], axis=0)
    a0 = jnp.dot(w0_ref[...], cols0, preferred_element_type=jnp.float32)
    h0 = jnp.where(live, jnp.maximum(a0 + t0_ref[...], 0.0),
                   0.0).astype(jnp.bfloat16)

    cols1 = jnp.concatenate([_lshift(h0, k - 2) for k in range(5)], axis=0)
    a1 = jnp.dot(w1_ref[...], cols1, preferred_element_type=jnp.float32)
    h1 = jnp.where(live, jnp.maximum(a1 + t1_ref[...], 0.0),
                   0.0).astype(jnp.bfloat16)

    cols2 = jnp.concatenate([_lshift(h1, k - 1) for k in range(3)], axis=0)
    a2 = jnp.dot(w2_ref[...], cols2, preferred_element_type=jnp.float32)
    h2 = jnp.maximum(a2 + t2_ref[...], 0.0).astype(jnp.bfloat16)

    pooled = jnp.dot(h2, pool_ref[...], preferred_element_type=jnp.float32)
    out = jax.lax.dot_general(pooled, fcw_ref[...], (((0,), (0,)), ((), ())),
                              preferred_element_type=jnp.float32)
    o_ref[0] = out + fcb_ref[...]


def kernel(conv0_w, conv0_scale, conv0_shift, conv1_w, conv1_scale,
           conv1_shift, conv2_w, conv2_scale, conv2_shift, fc_w, fc_b, x):
    N, cin, L = x.shape
    b_tile = 64
    num_tiles = pl.cdiv(N, b_tile)
    n_pad = num_tiles * b_tile
    NL = b_tile * _SLOT

    xt = jnp.transpose(x, (1, 0, 2))
    xt = jnp.pad(xt, ((0, 0), (0, n_pad - N), (_GAP, _GAP)))
    xg = xt.reshape(cin, n_pad * _SLOT).astype(jnp.bfloat16)

    w0 = conv0_w * conv0_scale[None, None, :]
    w0 = jnp.pad(w0, ((0, 0), (0, 16 - cin), (0, _LANES - w0.shape[2])))
    w0 = jnp.transpose(w0, (2, 0, 1)).reshape(_LANES, 7 * 16)
    w0 = w0.astype(jnp.bfloat16)
    t0 = jnp.pad(conv0_shift, (0, _LANES - conv0_shift.shape[0])).reshape(_LANES, 1)

    w1 = conv1_w * conv1_scale[None, None, :]
    w1 = jnp.pad(w1, ((0, 0), (0, _LANES - w1.shape[1]), (0, 0)))
    w1 = jnp.transpose(w1, (2, 0, 1)).reshape(_LANES, 5 * _LANES)
    w1 = w1.astype(jnp.bfloat16)
    t1 = conv1_shift.reshape(_LANES, 1)

    w2 = conv2_w * conv2_scale[None, None, :]
    w2 = jnp.transpose(w2, (2, 0, 1)).reshape(64, 3 * _LANES)
    w2 = w2.astype(jnp.bfloat16)
    t2 = conv2_shift.reshape(64, 1)

    ar = jnp.arange(NL)
    posv = ar % _SLOT
    sig = (posv >= _GAP) & (posv < _SLOT - _GAP)
    blk = ar // _SLOT
    pm = (sig[:, None] & (blk[:, None] == jnp.arange(b_tile)[None, :]))
    pm = (pm.astype(jnp.float32) / L).astype(jnp.bfloat16)

    fcw = jnp.pad(fc_w, ((0, 0), (0, 0))).astype(jnp.float32)
    fcb = fc_b.reshape(1, 2)

    consts = [w0, t0, w1, t1, w2, t2, pm, fcw, fcb]
    out = pl.pallas_call(
        _fcn_kernel,
        out_shape=jax.ShapeDtypeStruct((num_tiles, b_tile, 2), jnp.float32),
        grid=(num_tiles,),
        in_specs=[pl.BlockSpec((cin, NL), lambda n: (0, n))]
        + [pl.BlockSpec(a.shape, lambda n, nd=a.ndim: (0,) * nd) for a in consts],
        out_specs=pl.BlockSpec((1, b_tile, 2), lambda n: (n, 0, 0)),
        compiler_params=pltpu.CompilerParams(
            dimension_semantics=("parallel",)),
    )(xg, *consts)
    return out.reshape(n_pad, 2)[:N]


# speedup vs baseline: 4.5825x; 1.2652x over previous
import jax
import jax.numpy as jnp
from jax.experimental import pallas as pl
from jax.experimental.pallas import tpu as pltpu

_LANES = 128
_SLOT = 136
_GAP = 4


def _lshift(h, s):
    if s == 0:
        return h
    if s > 0:
        return jnp.pad(h[:, s:], ((0, 0), (0, s)))
    return jnp.pad(h[:, :s], ((0, 0), (-s, 0)))


def _fcn_kernel(x_ref, w0_ref, t0_ref, w1_ref, t1_ref, w2_ref, t2_ref,
                pool_ref, fcw_ref, fcb_ref, o_ref):
    NL = x_ref.shape[1]
    pos = jax.lax.broadcasted_iota(jnp.int32, (1, NL), 1) % _SLOT
    live = (pos >= _GAP) & (pos < _SLOT - _GAP)

    xb = jnp.pad(x_ref[...], ((0, 16 - x_ref.shape[0]), (0, 0)))
    cols0 = jnp.concatenate([_lshift(xb, k - 3) for k in range(7)], axis=0)
    a0 = jnp.dot(w0_ref[...], cols0, preferred_element_type=jnp.float32)
    h0 = jnp.where(live, jnp.maximum(a0 + t0_ref[...], 0.0),
                   0.0).astype(jnp.bfloat16)

    cols1 = jnp.concatenate([_lshift(h0, k - 2) for k in range(5)], axis=0)
    a1 = jnp.dot(w1_ref[...], cols1, preferred_element_type=jnp.float32)
    h1 = jnp.where(live, jnp.maximum(a1 + t1_ref[...], 0.0),
                   0.0).astype(jnp.bfloat16)

    cols2 = jnp.concatenate([_lshift(h1, k - 1) for k in range(3)], axis=0)
    a2 = jnp.dot(w2_ref[...], cols2, preferred_element_type=jnp.float32)
    h2 = jnp.maximum(a2 + t2_ref[...], 0.0).astype(jnp.bfloat16)

    pooled = jnp.dot(h2, pool_ref[...], preferred_element_type=jnp.float32)
    out = jax.lax.dot_general(pooled, fcw_ref[...], (((0,), (0,)), ((), ())),
                              preferred_element_type=jnp.float32)
    o_ref[0] = out + fcb_ref[...]


def kernel(conv0_w, conv0_scale, conv0_shift, conv1_w, conv1_scale,
           conv1_shift, conv2_w, conv2_scale, conv2_shift, fc_w, fc_b, x):
    N, cin, L = x.shape
    b_tile = 64
    num_tiles = pl.cdiv(N, b_tile)
    n_pad = num_tiles * b_tile
    NL = b_tile * _SLOT

    xt = jnp.transpose(x, (1, 0, 2))
    xt = jnp.pad(xt, ((0, 0), (0, n_pad - N), (_GAP, _GAP)))
    xg = xt.reshape(cin, n_pad * _SLOT).astype(jnp.bfloat16)

    c0out = conv0_w.shape[2]
    w0 = conv0_w * conv0_scale[None, None, :]
    w0 = jnp.pad(w0, ((0, 0), (0, 16 - cin), (0, 0)))
    w0 = jnp.transpose(w0, (2, 0, 1)).reshape(c0out, 7 * 16)
    w0 = w0.astype(jnp.bfloat16)
    t0 = conv0_shift.reshape(c0out, 1)

    w1 = conv1_w * conv1_scale[None, None, :]
    w1 = jnp.transpose(w1, (2, 0, 1)).reshape(_LANES, 5 * c0out)
    w1 = w1.astype(jnp.bfloat16)
    t1 = conv1_shift.reshape(_LANES, 1)

    w2 = conv2_w * conv2_scale[None, None, :]
    w2 = jnp.transpose(w2, (2, 0, 1)).reshape(64, 3 * _LANES)
    w2 = w2.astype(jnp.bfloat16)
    t2 = conv2_shift.reshape(64, 1)

    ar = jnp.arange(NL)
    posv = ar % _SLOT
    sig = (posv >= _GAP) & (posv < _SLOT - _GAP)
    blk = ar // _SLOT
    pm = (sig[:, None] & (blk[:, None] == jnp.arange(b_tile)[None, :]))
    pm = (pm.astype(jnp.float32) / L).astype(jnp.bfloat16)

    fcw = jnp.pad(fc_w, ((0, 0), (0, 0))).astype(jnp.float32)
    fcb = fc_b.reshape(1, 2)

    consts = [w0, t0, w1, t1, w2, t2, pm, fcw, fcb]
    out = pl.pallas_call(
        _fcn_kernel,
        out_shape=jax.ShapeDtypeStruct((num_tiles, b_tile, 2), jnp.float32),
        grid=(num_tiles,),
        in_specs=[pl.BlockSpec((cin, NL), lambda n: (0, n))]
        + [pl.BlockSpec(a.shape, lambda n, nd=a.ndim: (0,) * nd) for a in consts],
        out_specs=pl.BlockSpec((1, b_tile, 2), lambda n: (n, 0, 0)),
        compiler_params=pltpu.CompilerParams(
            dimension_semantics=("parallel",)),
    )(xg, *consts)
    return out.reshape(n_pad, 2)[:N]


# speedup vs baseline: 5.0096x; 1.0932x over previous
import jax
import jax.numpy as jnp
from jax.experimental import pallas as pl
from jax.experimental.pallas import tpu as pltpu

_LANES = 128
_SLOT = 136
_GAP = 4
_NEG = -1.0e30


def _lshift(h, s):
    if s == 0:
        return h
    if s > 0:
        return jnp.pad(h[:, s:], ((0, 0), (0, s)))
    return jnp.pad(h[:, :s], ((0, 0), (-s, 0)))


def _fcn_kernel(x_ref, w0_ref, w1_ref, w2_ref, pool_ref, fcw_ref, fcb_ref,
                o_ref):
    NL = x_ref.shape[1]
    pos = jax.lax.broadcasted_iota(jnp.int32, (2, NL), 1) % _SLOT
    live = (pos >= _GAP) & (pos < _SLOT - _GAP)
    sel = live ^ (jax.lax.broadcasted_iota(jnp.int32, (2, NL), 0) == 1)
    ind = jnp.where(sel, 1.0, 0.0).astype(jnp.bfloat16)

    xb = jnp.pad(x_ref[...], ((0, 16 - x_ref.shape[0]), (0, 0)))
    cols0 = jnp.concatenate([_lshift(xb, k - 3) for k in range(7)] + [ind],
                            axis=0)
    a0 = jnp.dot(w0_ref[...], cols0, preferred_element_type=jnp.float32)
    h0 = jnp.maximum(a0, 0.0).astype(jnp.bfloat16)

    cols1 = jnp.concatenate([_lshift(h0, k - 2) for k in range(5)] + [ind],
                            axis=0)
    a1 = jnp.dot(w1_ref[...], cols1, preferred_element_type=jnp.float32)
    h1 = jnp.maximum(a1, 0.0).astype(jnp.bfloat16)

    cols2 = jnp.concatenate([_lshift(h1, k - 1) for k in range(3)]
                            + [ind[:1]], axis=0)
    a2 = jnp.dot(w2_ref[...], cols2, preferred_element_type=jnp.float32)
    h2 = jnp.maximum(a2, 0.0).astype(jnp.bfloat16)

    pooled = jnp.dot(h2, pool_ref[...], preferred_element_type=jnp.float32)
    out = jax.lax.dot_general(pooled, fcw_ref[...], (((0,), (0,)), ((), ())),
                              preferred_element_type=jnp.float32)
    o_ref[0] = out + fcb_ref[...]


def kernel(conv0_w, conv0_scale, conv0_shift, conv1_w, conv1_scale,
           conv1_shift, conv2_w, conv2_scale, conv2_shift, fc_w, fc_b, x):
    N, cin, L = x.shape
    b_tile = 64
    num_tiles = pl.cdiv(N, b_tile)
    n_pad = num_tiles * b_tile
    NL = b_tile * _SLOT

    xt = jnp.transpose(x, (1, 0, 2))
    xt = jnp.pad(xt, ((0, 0), (0, n_pad - N), (_GAP, _GAP)))
    xg = xt.reshape(cin, n_pad * _SLOT).astype(jnp.bfloat16)

    def bias_gap_cols(w, t, neg_gap):
        c1 = t.reshape(-1, 1)
        cols = [w, c1]
        if neg_gap:
            cols.append(jnp.full_like(c1, _NEG))
        return jnp.concatenate(cols, axis=1)

    c0out = conv0_w.shape[2]
    w0 = conv0_w * conv0_scale[None, None, :]
    w0 = jnp.pad(w0, ((0, 0), (0, 16 - cin), (0, 0)))
    w0 = jnp.transpose(w0, (2, 0, 1)).reshape(c0out, 7 * 16)
    w0 = bias_gap_cols(w0, conv0_shift, True).astype(jnp.bfloat16)

    w1 = conv1_w * conv1_scale[None, None, :]
    w1 = jnp.transpose(w1, (2, 0, 1)).reshape(_LANES, 5 * c0out)
    w1 = bias_gap_cols(w1, conv1_shift, True).astype(jnp.bfloat16)

    w2 = conv2_w * conv2_scale[None, None, :]
    w2 = jnp.transpose(w2, (2, 0, 1)).reshape(64, 3 * _LANES)
    w2 = bias_gap_cols(w2, conv2_shift, False).astype(jnp.bfloat16)

    ar = jnp.arange(NL)
    posv = ar % _SLOT
    sig = (posv >= _GAP) & (posv < _SLOT - _GAP)
    blk = ar // _SLOT
    pm = (sig[:, None] & (blk[:, None] == jnp.arange(b_tile)[None, :]))
    pm = (pm.astype(jnp.float32) / L).astype(jnp.bfloat16)

    fcw = fc_w.astype(jnp.float32)
    fcb = fc_b.reshape(1, 2)

    consts = [w0, w1, w2, pm, fcw, fcb]
    out = pl.pallas_call(
        _fcn_kernel,
        out_shape=jax.ShapeDtypeStruct((num_tiles, b_tile, 2), jnp.float32),
        grid=(num_tiles,),
        in_specs=[pl.BlockSpec((cin, NL), lambda n: (0, n))]
        + [pl.BlockSpec(a.shape, lambda n, nd=a.ndim: (0,) * nd) for a in consts],
        out_specs=pl.BlockSpec((1, b_tile, 2), lambda n: (n, 0, 0)),
        compiler_params=pltpu.CompilerParams(
            dimension_semantics=("parallel",)),
    )(xg, *consts)
    return out.reshape(n_pad, 2)[:N]


# speedup vs baseline: 5.2550x; 1.0490x over previous
import jax
import jax.numpy as jnp
from jax.experimental import pallas as pl
from jax.experimental.pallas import tpu as pltpu

_LANES = 128
_SLOT = 136
_GAP = 4
_NEG = -1.0e30


def _lshift(h, s):
    if s == 0:
        return h
    if s > 0:
        return jnp.pad(h[:, s:], ((0, 0), (0, s)))
    return jnp.pad(h[:, :s], ((0, 0), (-s, 0)))


def _fcn_kernel(x_ref, w0_ref, w1_ref, w2_ref, pool_ref, fcw_ref, fcb_ref,
                o_ref):
    NL = x_ref.shape[1]
    pos = jax.lax.broadcasted_iota(jnp.int32, (2, NL), 1) % _SLOT
    live = (pos >= _GAP) & (pos < _SLOT - _GAP)
    sel = live ^ (jax.lax.broadcasted_iota(jnp.int32, (2, NL), 0) == 1)
    ind = jnp.where(sel, 1.0, 0.0).astype(jnp.bfloat16)

    xb = jnp.pad(x_ref[...], ((0, 16 - x_ref.shape[0]), (0, 0)))
    cols0 = jnp.concatenate([_lshift(xb, k - 3) for k in range(7)] + [ind],
                            axis=0)
    a0 = jnp.dot(w0_ref[...], cols0, preferred_element_type=jnp.float32)
    h0 = jnp.maximum(a0, 0.0).astype(jnp.bfloat16)

    cols1 = jnp.concatenate([_lshift(h0, k - 2) for k in range(5)] + [ind],
                            axis=0)
    a1 = jnp.dot(w1_ref[...], cols1, preferred_element_type=jnp.float32)
    h1 = jnp.maximum(a1, 0.0).astype(jnp.bfloat16)

    cols2 = jnp.concatenate([_lshift(h1, k - 1) for k in range(3)]
                            + [ind[:1]], axis=0)
    a2 = jnp.dot(w2_ref[...], cols2, preferred_element_type=jnp.float32)
    h2 = jnp.maximum(a2, 0.0).astype(jnp.bfloat16)

    pooled = jnp.dot(h2, pool_ref[...], preferred_element_type=jnp.float32)
    out = jax.lax.dot_general(pooled, fcw_ref[...], (((0,), (0,)), ((), ())),
                              preferred_element_type=jnp.float32)
    o_ref[0] = out + fcb_ref[...]


def kernel(conv0_w, conv0_scale, conv0_shift, conv1_w, conv1_scale,
           conv1_shift, conv2_w, conv2_scale, conv2_shift, fc_w, fc_b, x):
    N, cin, L = x.shape
    b_tile = 128
    num_tiles = pl.cdiv(N, b_tile)
    n_pad = num_tiles * b_tile
    NL = b_tile * _SLOT

    xt = jnp.transpose(x, (1, 0, 2))
    xt = jnp.pad(xt, ((0, 0), (0, n_pad - N), (_GAP, _GAP)))
    xg = xt.reshape(cin, n_pad * _SLOT).astype(jnp.bfloat16)

    def bias_gap_cols(w, t, neg_gap):
        c1 = t.reshape(-1, 1)
        cols = [w, c1]
        if neg_gap:
            cols.append(jnp.full_like(c1, _NEG))
        return jnp.concatenate(cols, axis=1)

    c0out = conv0_w.shape[2]
    w0 = conv0_w * conv0_scale[None, None, :]
    w0 = jnp.pad(w0, ((0, 0), (0, 16 - cin), (0, 0)))
    w0 = jnp.transpose(w0, (2, 0, 1)).reshape(c0out, 7 * 16)
    w0 = bias_gap_cols(w0, conv0_shift, True).astype(jnp.bfloat16)

    w1 = conv1_w * conv1_scale[None, None, :]
    w1 = jnp.transpose(w1, (2, 0, 1)).reshape(_LANES, 5 * c0out)
    w1 = bias_gap_cols(w1, conv1_shift, True).astype(jnp.bfloat16)

    w2 = conv2_w * conv2_scale[None, None, :]
    w2 = jnp.transpose(w2, (2, 0, 1)).reshape(64, 3 * _LANES)
    w2 = bias_gap_cols(w2, conv2_shift, False).astype(jnp.bfloat16)

    ar = jnp.arange(NL)
    posv = ar % _SLOT
    sig = (posv >= _GAP) & (posv < _SLOT - _GAP)
    blk = ar // _SLOT
    pm = (sig[:, None] & (blk[:, None] == jnp.arange(b_tile)[None, :]))
    pm = (pm.astype(jnp.float32) / L).astype(jnp.bfloat16)

    fcw = fc_w.astype(jnp.float32)
    fcb = fc_b.reshape(1, 2)

    consts = [w0, w1, w2, pm, fcw, fcb]
    out = pl.pallas_call(
        _fcn_kernel,
        out_shape=jax.ShapeDtypeStruct((num_tiles, b_tile, 2), jnp.float32),
        grid=(num_tiles,),
        in_specs=[pl.BlockSpec((cin, NL), lambda n: (0, n))]
        + [pl.BlockSpec(a.shape, lambda n, nd=a.ndim: (0,) * nd) for a in consts],
        out_specs=pl.BlockSpec((1, b_tile, 2), lambda n: (n, 0, 0)),
        compiler_params=pltpu.CompilerParams(
            dimension_semantics=("parallel",)),
    )(xg, *consts)
    return out.reshape(n_pad, 2)[:N]


# speedup vs baseline: 5.3916x; 1.0260x over previous
import jax
import jax.numpy as jnp
from jax.experimental import pallas as pl
from jax.experimental.pallas import tpu as pltpu

_LANES = 128
_SLOT = 136
_GAP = 4
_NEG = -1.0e30


def _lshift(h, s):
    if s == 0:
        return h
    if s > 0:
        return jnp.pad(h[:, s:], ((0, 0), (0, s)))
    return jnp.pad(h[:, :s], ((0, 0), (-s, 0)))


def _chain(xh, ind, w0, w1, w2, pmh):
    cols0 = jnp.concatenate([_lshift(xh, k - 3) for k in range(7)] + [ind],
                            axis=0)
    a0 = jnp.dot(w0, cols0, preferred_element_type=jnp.float32)
    h0 = jnp.maximum(a0, 0.0).astype(jnp.bfloat16)

    cols1 = jnp.concatenate([_lshift(h0, k - 2) for k in range(5)] + [ind],
                            axis=0)
    a1 = jnp.dot(w1, cols1, preferred_element_type=jnp.float32)
    h1 = jnp.maximum(a1, 0.0).astype(jnp.bfloat16)

    cols2 = jnp.concatenate([_lshift(h1, k - 1) for k in range(3)]
                            + [ind[:1]], axis=0)
    a2 = jnp.dot(w2, cols2, preferred_element_type=jnp.float32)
    h2 = jnp.maximum(a2, 0.0).astype(jnp.bfloat16)
    return jnp.dot(h2, pmh, preferred_element_type=jnp.float32)


def _fcn_kernel(x_ref, w0_ref, w1_ref, w2_ref, pool_ref, fcw_ref, fcb_ref,
                o_ref):
    NL = x_ref.shape[1]
    half = NL // 2
    pos = jax.lax.broadcasted_iota(jnp.int32, (2, half), 1) % _SLOT
    live = (pos >= _GAP) & (pos < _SLOT - _GAP)
    sel = live ^ (jax.lax.broadcasted_iota(jnp.int32, (2, half), 0) == 1)
    ind = jnp.where(sel, 1.0, 0.0).astype(jnp.bfloat16)

    xb = jnp.pad(x_ref[...], ((0, 16 - x_ref.shape[0]), (0, 0)))
    pooled = jnp.concatenate(
        [_chain(xb[:, :half], ind, w0_ref[...], w1_ref[...], w2_ref[...],
                pool_ref[...]),
         _chain(xb[:, half:], ind, w0_ref[...], w1_ref[...], w2_ref[...],
                pool_ref[...])], axis=1)
    out = jax.lax.dot_general(pooled, fcw_ref[...], (((0,), (0,)), ((), ())),
                              preferred_element_type=jnp.float32)
    o_ref[0] = out + fcb_ref[...]


def kernel(conv0_w, conv0_scale, conv0_shift, conv1_w, conv1_scale,
           conv1_shift, conv2_w, conv2_scale, conv2_shift, fc_w, fc_b, x):
    N, cin, L = x.shape
    b_tile = 128
    num_tiles = pl.cdiv(N, b_tile)
    n_pad = num_tiles * b_tile
    NL = b_tile * _SLOT

    xt = jnp.transpose(x, (1, 0, 2))
    xt = jnp.pad(xt, ((0, 0), (0, n_pad - N), (_GAP, _GAP)))
    xg = xt.reshape(cin, n_pad * _SLOT).astype(jnp.bfloat16)

    def bias_gap_cols(w, t, neg_gap):
        c1 = t.reshape(-1, 1)
        cols = [w, c1]
        if neg_gap:
            cols.append(jnp.full_like(c1, _NEG))
        return jnp.concatenate(cols, axis=1)

    c0out = conv0_w.shape[2]
    w0 = conv0_w * conv0_scale[None, None, :]
    w0 = jnp.pad(w0, ((0, 0), (0, 16 - cin), (0, 0)))
    w0 = jnp.transpose(w0, (2, 0, 1)).reshape(c0out, 7 * 16)
    w0 = bias_gap_cols(w0, conv0_shift, True).astype(jnp.bfloat16)

    w1 = conv1_w * conv1_scale[None, None, :]
    w1 = jnp.transpose(w1, (2, 0, 1)).reshape(_LANES, 5 * c0out)
    w1 = bias_gap_cols(w1, conv1_shift, True).astype(jnp.bfloat16)

    w2 = conv2_w * conv2_scale[None, None, :]
    w2 = jnp.transpose(w2, (2, 0, 1)).reshape(64, 3 * _LANES)
    w2 = bias_gap_cols(w2, conv2_shift, False).astype(jnp.bfloat16)

    ar = jnp.arange(NL // 2)
    posv = ar % _SLOT
    sig = (posv >= _GAP) & (posv < _SLOT - _GAP)
    blk = ar // _SLOT
    pm = (sig[:, None] & (blk[:, None] == jnp.arange(b_tile // 2)[None, :]))
    pm = (pm.astype(jnp.float32) / L).astype(jnp.bfloat16)

    fcw = fc_w.astype(jnp.float32)
    fcb = fc_b.reshape(1, 2)

    consts = [w0, w1, w2, pm, fcw, fcb]
    out = pl.pallas_call(
        _fcn_kernel,
        out_shape=jax.ShapeDtypeStruct((num_tiles, b_tile, 2), jnp.float32),
        grid=(num_tiles,),
        in_specs=[pl.BlockSpec((cin, NL), lambda n: (0, n))]
        + [pl.BlockSpec(a.shape, lambda n, nd=a.ndim: (0,) * nd) for a in consts],
        out_specs=pl.BlockSpec((1, b_tile, 2), lambda n: (n, 0, 0)),
        compiler_params=pltpu.CompilerParams(
            dimension_semantics=("parallel",)),
    )(xg, *consts)
    return out.reshape(n_pad, 2)[:N]


# speedup vs baseline: 5.6638x; 1.0505x over previous
import jax
import jax.numpy as jnp
from jax.experimental import pallas as pl
from jax.experimental.pallas import tpu as pltpu

_LANES = 128
_SLOT = 136
_GAP = 4
_NEG = -1.0e30
_CHAINS = 4


def _lshift(h, s):
    if s == 0:
        return h
    if s > 0:
        return jnp.pad(h[:, s:], ((0, 0), (0, s)))
    return jnp.pad(h[:, :s], ((0, 0), (-s, 0)))


def _chain(xh, ind, w0, w1, w2, pmh):
    cols0 = jnp.concatenate([_lshift(xh, k - 3) for k in range(7)] + [ind],
                            axis=0)
    a0 = jnp.dot(w0, cols0, preferred_element_type=jnp.float32)
    h0 = jnp.maximum(a0, 0.0).astype(jnp.bfloat16)

    cols1 = jnp.concatenate([_lshift(h0, k - 2) for k in range(5)] + [ind],
                            axis=0)
    a1 = jnp.dot(w1, cols1, preferred_element_type=jnp.float32)
    h1 = jnp.maximum(a1, 0.0).astype(jnp.bfloat16)

    cols2 = jnp.concatenate([_lshift(h1, k - 1) for k in range(3)]
                            + [ind[:1]], axis=0)
    a2 = jnp.dot(w2, cols2, preferred_element_type=jnp.float32)
    h2 = jnp.maximum(a2, 0.0).astype(jnp.bfloat16)
    return jnp.dot(h2, pmh, preferred_element_type=jnp.float32)


def _fcn_kernel(x_ref, w0_ref, w1_ref, w2_ref, pool_ref, fcw_ref, fcb_ref,
                o_ref):
    NL = x_ref.shape[1]
    half = NL // _CHAINS
    pos = jax.lax.broadcasted_iota(jnp.int32, (2, half), 1) % _SLOT
    live = (pos >= _GAP) & (pos < _SLOT - _GAP)
    sel = live ^ (jax.lax.broadcasted_iota(jnp.int32, (2, half), 0) == 1)
    ind = jnp.where(sel, 1.0, 0.0).astype(jnp.bfloat16)

    xb = jnp.pad(x_ref[...], ((0, 16 - x_ref.shape[0]), (0, 0)))
    pooled = jnp.concatenate(
        [_chain(xb[:, i * half:(i + 1) * half], ind, w0_ref[...], w1_ref[...],
                w2_ref[...], pool_ref[...]) for i in range(_CHAINS)], axis=1)
    out = jax.lax.dot_general(pooled, fcw_ref[...], (((0,), (0,)), ((), ())),
                              preferred_element_type=jnp.float32)
    o_ref[0] = out + fcb_ref[...]


def kernel(conv0_w, conv0_scale, conv0_shift, conv1_w, conv1_scale,
           conv1_shift, conv2_w, conv2_scale, conv2_shift, fc_w, fc_b, x):
    N, cin, L = x.shape
    b_tile = 256
    num_tiles = pl.cdiv(N, b_tile)
    n_pad = num_tiles * b_tile
    NL = b_tile * _SLOT

    xt = jnp.transpose(x, (1, 0, 2))
    xt = jnp.pad(xt, ((0, 0), (0, n_pad - N), (_GAP, _GAP)))
    xg = xt.reshape(cin, n_pad * _SLOT).astype(jnp.bfloat16)

    def bias_gap_cols(w, t, neg_gap):
        c1 = t.reshape(-1, 1)
        cols = [w, c1]
        if neg_gap:
            cols.append(jnp.full_like(c1, _NEG))
        return jnp.concatenate(cols, axis=1)

    c0out = conv0_w.shape[2]
    w0 = conv0_w * conv0_scale[None, None, :]
    w0 = jnp.pad(w0, ((0, 0), (0, 16 - cin), (0, 0)))
    w0 = jnp.transpose(w0, (2, 0, 1)).reshape(c0out, 7 * 16)
    w0 = bias_gap_cols(w0, conv0_shift, True).astype(jnp.bfloat16)

    w1 = conv1_w * conv1_scale[None, None, :]
    w1 = jnp.transpose(w1, (2, 0, 1)).reshape(_LANES, 5 * c0out)
    w1 = bias_gap_cols(w1, conv1_shift, True).astype(jnp.bfloat16)

    w2 = conv2_w * conv2_scale[None, None, :]
    w2 = jnp.transpose(w2, (2, 0, 1)).reshape(64, 3 * _LANES)
    w2 = bias_gap_cols(w2, conv2_shift, False).astype(jnp.bfloat16)

    ar = jnp.arange(NL // _CHAINS)
    posv = ar % _SLOT
    sig = (posv >= _GAP) & (posv < _SLOT - _GAP)
    blk = ar // _SLOT
    pm = (sig[:, None] & (blk[:, None] == jnp.arange(b_tile // _CHAINS)[None, :]))
    pm = (pm.astype(jnp.float32) / L).astype(jnp.bfloat16)

    fcw = fc_w.astype(jnp.float32)
    fcb = fc_b.reshape(1, 2)

    consts = [w0, w1, w2, pm, fcw, fcb]
    out = pl.pallas_call(
        _fcn_kernel,
        out_shape=jax.ShapeDtypeStruct((num_tiles, b_tile, 2), jnp.float32),
        grid=(num_tiles,),
        in_specs=[pl.BlockSpec((cin, NL), lambda n: (0, n))]
        + [pl.BlockSpec(a.shape, lambda n, nd=a.ndim: (0,) * nd) for a in consts],
        out_specs=pl.BlockSpec((1, b_tile, 2), lambda n: (n, 0, 0)),
        compiler_params=pltpu.CompilerParams(
            dimension_semantics=("parallel",)),
    )(xg, *consts)
    return out.reshape(n_pad, 2)[:N]
